# baseline (speedup 1.0000x reference)
import numpy as np

import jax
import jax.numpy as jnp
from jax.experimental import pallas as pl
from jax.experimental.pallas import tpu as pltpu
from jax.sharding import Mesh, PartitionSpec as P

S_SRC = 256
S_TRG = 128
IN_DIM, OUT_DIM, EMB, HID = 50, 50, 32, 64
G4 = 4 * HID
B_BLK = 128
CH = 8

_INTERPRET = False


def _seq2seq_kernel(src_ref, tok0_ref, embT_ref, encW_ref, benc_ref,
                    we_ref, attnb_ref, wh_ref, v_ref, dembT_ref, decW_ref,
                    bdec_ref, fcw_ref, fcb_ref, out_ref, es_ref, ps_ref,
                    ps2_ref):
    f32 = jnp.float32
    iota64 = jax.lax.broadcasted_iota(jnp.int32, (64, B_BLK), 0)
    h0 = jnp.zeros((HID, B_BLK), f32)

    def lstm_gates(g, c):
        i_ = jax.nn.sigmoid(g[0:64])
        f_ = jax.nn.sigmoid(g[64:128])
        g_ = jnp.tanh(g[128:192])
        o_ = jax.nn.sigmoid(g[192:256])
        c = f_ * c + i_ * g_
        h = o_ * jnp.tanh(c)
        return h, c

    def enc_body(t, carry):
        h, c = carry
        oh = (iota64 == src_ref[t]).astype(f32)
        x = jnp.dot(embT_ref[...], oh, preferred_element_type=f32)
        xh = jnp.concatenate([x, h], axis=0)
        g = jnp.dot(encW_ref[...], xh, preferred_element_type=f32) + benc_ref[...]
        h, c = lstm_gates(g, c)
        es_ref[t] = h
        ps_ref[t] = jnp.dot(we_ref[...], h, preferred_element_type=f32) + attnb_ref[...]
        return h, c

    h, c = jax.lax.fori_loop(0, S_SRC, enc_body, (h0, h0))

    for k0 in range(0, HID, 8):
        ps2_ref[k0:k0 + 8] = jnp.transpose(
            ps_ref[:, k0:k0 + 8, :], (1, 0, 2)).astype(jnp.bfloat16)

    def dec_body(t, carry):
        h, c, tok = carry
        oh = (iota64 == tok).astype(f32)
        e = jnp.dot(dembT_ref[...], oh, preferred_element_type=f32)
        q3 = jnp.dot(wh_ref[...], h,
                     preferred_element_type=f32).reshape(HID, 1, B_BLK).astype(jnp.bfloat16)
        v3 = v_ref[...]
        sc = jnp.zeros((S_SRC, B_BLK), jnp.bfloat16)
        for g in range(0, HID, 8):
            tm = [v3[g + j] * jnp.tanh(q3[g + j] + ps2_ref[g + j])
                  for j in range(8)]
            sc = sc + (((tm[0] + tm[1]) + (tm[2] + tm[3]))
                       + ((tm[4] + tm[5]) + (tm[6] + tm[7])))
        scf = sc.astype(f32)
        m = jnp.max(scf, axis=0, keepdims=True)
        ex = jnp.exp(scf - m)
        l = jnp.sum(ex, axis=0, keepdims=True)
        a3 = (ex / l).reshape(S_SRC, 1, B_BLK)
        parts = []
        for j in range(4):
            acc = jnp.zeros((HID, B_BLK), f32)
            for s in range(j * 64, (j + 1) * 64):
                acc = acc + a3[s] * es_ref[s]
            parts.append(acc)
        ctx = (parts[0] + parts[1]) + (parts[2] + parts[3])
        x = jnp.concatenate([e, ctx, h], axis=0)
        g = jnp.dot(decW_ref[...], x, preferred_element_type=f32) + bdec_ref[...]
        h, c = lstm_gates(g, c)
        pred = jnp.dot(fcw_ref[...], h, preferred_element_type=f32) + fcb_ref[...]
        out_ref[t] = pred[0:OUT_DIM]
        mx = jnp.max(pred, axis=0, keepdims=True)
        tok = jnp.min(jnp.where(pred == mx, iota64, jnp.int32(63)),
                      axis=0, keepdims=True)
        return h, c, tok

    jax.lax.fori_loop(0, S_TRG - 1, dec_body, (h, c, tok0_ref[...]))


def kernel(src, trg, enc_emb, enc_Wih, enc_Whh, enc_bih, enc_bhh, attn_W,
           attn_b, v_w, dec_emb, dec_Wih, dec_Whh, dec_bih, dec_bhh,
           fc_W, fc_b):
    f32 = jnp.float32
    b = src.shape[0]
    srcT = src.T.reshape(S_SRC, 1, b)
    tok0 = trg[:, 0].reshape(1, b)
    embT = jnp.zeros((EMB, 64), f32).at[:, :IN_DIM].set(enc_emb.T)
    dembT = jnp.zeros((EMB, 64), f32).at[:, :OUT_DIM].set(dec_emb.T)
    encW = jnp.concatenate([enc_Wih, enc_Whh], axis=1)
    benc = jnp.broadcast_to((enc_bih + enc_bhh)[:, None], (G4, B_BLK))
    W_h, W_e = attn_W[:, :HID], attn_W[:, HID:]
    attnbB = jnp.broadcast_to(attn_b[:, None], (HID, B_BLK))
    vB = jnp.broadcast_to(
        v_w.astype(jnp.bfloat16)[:, None, None], (HID, 1, B_BLK))
    decW = jnp.concatenate([dec_Wih, dec_Whh], axis=1)
    bdec = jnp.broadcast_to((dec_bih + dec_bhh)[:, None], (G4, B_BLK))
    fcW64 = jnp.zeros((64, HID), f32).at[:OUT_DIM].set(fc_W)
    fcb64 = jnp.full((64,), -1e30, f32).at[:OUT_DIM].set(fc_b)
    fcbB = jnp.broadcast_to(fcb64[:, None], (64, B_BLK))

    def full(shape):
        return pl.BlockSpec(shape, lambda i: tuple(0 for _ in shape))

    def run_blocks(srcT_l, tok0_l, *weights):
        b_l = srcT_l.shape[-1]
        grid = (b_l // B_BLK,)
        in_specs = [
            pl.BlockSpec((S_SRC, 1, B_BLK), lambda i: (0, 0, i)),
            pl.BlockSpec((1, B_BLK), lambda i: (0, i)),
            full((EMB, 64)), full((G4, 96)), full((G4, B_BLK)),
            full((HID, HID)), full((HID, B_BLK)), full((HID, HID)),
            full((HID, 1, B_BLK)), full((EMB, 64)), full((G4, 160)),
            full((G4, B_BLK)), full((64, HID)), full((64, B_BLK)),
        ]
        out_specs = pl.BlockSpec((S_TRG - 1, OUT_DIM, B_BLK),
                                 lambda i: (0, 0, i))
        return pl.pallas_call(
            _seq2seq_kernel,
            out_shape=jax.ShapeDtypeStruct((S_TRG - 1, OUT_DIM, b_l), f32),
            grid=grid,
            in_specs=in_specs,
            out_specs=out_specs,
            scratch_shapes=[
                pltpu.VMEM((S_SRC, HID, B_BLK), f32),
                pltpu.VMEM((S_SRC, HID, B_BLK), f32),
                pltpu.VMEM((HID, S_SRC, B_BLK), jnp.bfloat16),
            ],
            compiler_params=pltpu.CompilerParams(
                dimension_semantics=("parallel",),
                vmem_limit_bytes=50 * 1024 * 1024,
            ),
            name="seq2seq_fused",
            interpret=_INTERPRET,
        )(srcT_l, tok0_l, *weights)

    args = (srcT, tok0, embT, encW, benc, W_e, attnbB, W_h, vB, dembT,
            decW, bdec, fcW64, fcbB)
    devs = jax.devices()
    n_dev = 2 if len(devs) >= 2 and b % (2 * B_BLK) == 0 else 1
    if n_dev == 2:
        mesh = Mesh(np.asarray(devs[:2]), ("d",))
        w_specs = tuple(P() for _ in range(len(args) - 2))
        predsT = jax.shard_map(
            run_blocks, mesh=mesh,
            in_specs=(P(None, None, "d"), P(None, "d")) + w_specs,
            out_specs=P(None, None, "d"),
            check_vma=False,
        )(*args)
    else:
        predsT = run_blocks(*args)
    preds = predsT.transpose(2, 0, 1)
    return jnp.concatenate([jnp.zeros((b, 1, OUT_DIM), f32), preds], axis=1)


# speedup vs baseline: 1.1763x; 1.1763x over previous
import numpy as np

import jax
import jax.numpy as jnp
from jax.experimental import pallas as pl
from jax.experimental.pallas import tpu as pltpu
from jax.sharding import Mesh, PartitionSpec as P

S_SRC = 256
S_TRG = 128
IN_DIM, OUT_DIM, EMB, HID = 50, 50, 32, 64
G4 = 4 * HID
W_BLK = 256
HB = 128

_INTERPRET = False


def _seq2seq_kernel(src_ref, tok0_ref, embT_ref, encW_ref, benc_ref,
                    we_ref, attnb_ref, wh_ref, v_ref, dembT_ref, decW_ref,
                    bdec_ref, fcw_ref, fcb_ref, out_ref, es_ref, ps_ref,
                    ps2_ref):
    f32 = jnp.float32
    bf16 = jnp.bfloat16
    iota64 = jax.lax.broadcasted_iota(jnp.int32, (64, W_BLK), 0)
    h0 = jnp.zeros((HID, W_BLK), f32)

    def lstm_gates(g, c):
        i_ = jax.nn.sigmoid(g[0:64])
        f_ = jax.nn.sigmoid(g[64:128])
        g_ = jnp.tanh(g[128:192])
        o_ = jax.nn.sigmoid(g[192:256])
        c = f_ * c + i_ * g_
        h = o_ * jnp.tanh(c)
        return h, c

    def enc_body(t, carry):
        h, c = carry
        oh = (iota64 == src_ref[t]).astype(f32)
        x = jnp.dot(embT_ref[...], oh, preferred_element_type=f32)
        xh = jnp.concatenate([x, h], axis=0)
        g = jnp.dot(encW_ref[...], xh, preferred_element_type=f32) + benc_ref[...]
        h, c = lstm_gates(g, c)
        es_ref[t] = h
        ps_ref[t] = (jnp.dot(we_ref[...], h, preferred_element_type=f32)
                     + attnb_ref[...]).astype(bf16)
        return h, c

    h, c = jax.lax.fori_loop(0, S_SRC, enc_body, (h0, h0))

    for k0 in range(0, HID, 8):
        ps2_ref[k0:k0 + 8] = jnp.transpose(ps_ref[:, k0:k0 + 8, :], (1, 0, 2))

    def dec_body(t, carry):
        h, c, tok = carry
        oh = (iota64 == tok).astype(f32)
        e = jnp.dot(dembT_ref[...], oh, preferred_element_type=f32)
        q = jnp.dot(wh_ref[...], h, preferred_element_type=f32)
        v3 = v_ref[...]
        ctxs = []
        for lo in (0, HB):
            q3 = q[:, lo:lo + HB].reshape(HID, 1, HB).astype(bf16)
            sc = jnp.zeros((S_SRC, HB), bf16)
            for g8 in range(0, HID, 8):
                tm = [v3[g8 + j] * jnp.tanh(q3[g8 + j]
                                            + ps2_ref[g8 + j, :, lo:lo + HB])
                      for j in range(8)]
                sc = sc + (((tm[0] + tm[1]) + (tm[2] + tm[3]))
                           + ((tm[4] + tm[5]) + (tm[6] + tm[7])))
            scf = sc.astype(f32)
            m = jnp.max(scf, axis=0, keepdims=True)
            ex = jnp.exp(scf - m)
            l = jnp.sum(ex, axis=0, keepdims=True)
            a3 = (ex / l).reshape(S_SRC, 1, HB)
            parts = []
            for j in range(4):
                acc = jnp.zeros((HID, HB), f32)
                for s in range(j * 64, (j + 1) * 64):
                    acc = acc + a3[s] * es_ref[s, :, lo:lo + HB]
                parts.append(acc)
            ctxs.append((parts[0] + parts[1]) + (parts[2] + parts[3]))
        ctx = jnp.concatenate(ctxs, axis=1)
        x = jnp.concatenate([e, ctx, h], axis=0)
        g = jnp.dot(decW_ref[...], x, preferred_element_type=f32) + bdec_ref[...]
        h, c = lstm_gates(g, c)
        pred = jnp.dot(fcw_ref[...], h, preferred_element_type=f32) + fcb_ref[...]
        out_ref[t] = pred[0:OUT_DIM]
        mx = jnp.max(pred, axis=0, keepdims=True)
        tok = jnp.min(jnp.where(pred == mx, iota64, jnp.int32(63)),
                      axis=0, keepdims=True)
        return h, c, tok

    jax.lax.fori_loop(0, S_TRG - 1, dec_body, (h, c, tok0_ref[...]))


def kernel(src, trg, enc_emb, enc_Wih, enc_Whh, enc_bih, enc_bhh, attn_W,
           attn_b, v_w, dec_emb, dec_Wih, dec_Whh, dec_bih, dec_bhh,
           fc_W, fc_b):
    f32 = jnp.float32
    b = src.shape[0]
    srcT = src.T.reshape(S_SRC, 1, b)
    tok0 = trg[:, 0].reshape(1, b)
    embT = jnp.zeros((EMB, 64), f32).at[:, :IN_DIM].set(enc_emb.T)
    dembT = jnp.zeros((EMB, 64), f32).at[:, :OUT_DIM].set(dec_emb.T)
    encW = jnp.concatenate([enc_Wih, enc_Whh], axis=1)
    benc = jnp.broadcast_to((enc_bih + enc_bhh)[:, None], (G4, W_BLK))
    W_h, W_e = attn_W[:, :HID], attn_W[:, HID:]
    attnbB = jnp.broadcast_to(attn_b[:, None], (HID, W_BLK))
    vB = jnp.broadcast_to(
        v_w.astype(jnp.bfloat16)[:, None, None], (HID, 1, HB))
    decW = jnp.concatenate([dec_Wih, dec_Whh], axis=1)
    bdec = jnp.broadcast_to((dec_bih + dec_bhh)[:, None], (G4, W_BLK))
    fcW64 = jnp.zeros((64, HID), f32).at[:OUT_DIM].set(fc_W)
    fcb64 = jnp.full((64,), -1e30, f32).at[:OUT_DIM].set(fc_b)
    fcbB = jnp.broadcast_to(fcb64[:, None], (64, W_BLK))

    def full(shape):
        return pl.BlockSpec(shape, lambda i: tuple(0 for _ in shape))

    def run_blocks(srcT_l, tok0_l, *weights):
        b_l = srcT_l.shape[-1]
        grid = (b_l // W_BLK,)
        in_specs = [
            pl.BlockSpec((S_SRC, 1, W_BLK), lambda i: (0, 0, i)),
            pl.BlockSpec((1, W_BLK), lambda i: (0, i)),
            full((EMB, 64)), full((G4, 96)), full((G4, W_BLK)),
            full((HID, HID)), full((HID, W_BLK)), full((HID, HID)),
            full((HID, 1, HB)), full((EMB, 64)), full((G4, 160)),
            full((G4, W_BLK)), full((64, HID)), full((64, W_BLK)),
        ]
        out_specs = pl.BlockSpec((S_TRG - 1, OUT_DIM, W_BLK),
                                 lambda i: (0, 0, i))
        return pl.pallas_call(
            _seq2seq_kernel,
            out_shape=jax.ShapeDtypeStruct((S_TRG - 1, OUT_DIM, b_l), f32),
            grid=grid,
            in_specs=in_specs,
            out_specs=out_specs,
            scratch_shapes=[
                pltpu.VMEM((S_SRC, HID, W_BLK), f32),
                pltpu.VMEM((S_SRC, HID, W_BLK), jnp.bfloat16),
                pltpu.VMEM((HID, S_SRC, W_BLK), jnp.bfloat16),
            ],
            compiler_params=pltpu.CompilerParams(
                dimension_semantics=("parallel",),
                vmem_limit_bytes=55 * 1024 * 1024,
            ),
            name="seq2seq_fused",
            interpret=_INTERPRET,
        )(srcT_l, tok0_l, *weights)

    args = (srcT, tok0, embT, encW, benc, W_e, attnbB, W_h, vB, dembT,
            decW, bdec, fcW64, fcbB)
    devs = jax.devices()
    n_dev = 2 if len(devs) >= 2 and b % (2 * W_BLK) == 0 else 1
    if n_dev == 2:
        mesh = Mesh(np.asarray(devs[:2]), ("d",))
        w_specs = tuple(P() for _ in range(len(args) - 2))
        predsT = jax.shard_map(
            run_blocks, mesh=mesh,
            in_specs=(P(None, None, "d"), P(None, "d")) + w_specs,
            out_specs=P(None, None, "d"),
            check_vma=False,
        )(*args)
    else:
        predsT = run_blocks(*args)
    preds = predsT.transpose(2, 0, 1)
    return jnp.concatenate([jnp.zeros((b, 1, OUT_DIM), f32), preds], axis=1)


# speedup vs baseline: 1.1869x; 1.0090x over previous
import numpy as np

import jax
import jax.numpy as jnp
from jax.experimental import pallas as pl
from jax.experimental.pallas import tpu as pltpu
from jax.sharding import Mesh, PartitionSpec as P

S_SRC = 256
S_TRG = 128
IN_DIM, OUT_DIM, EMB, HID = 50, 50, 32, 64
G4 = 4 * HID
W_BLK = 256
HB = 128

_INTERPRET = False


def _seq2seq_kernel(src_ref, tok0_ref, embT_ref, encW_ref, benc_ref,
                    we_ref, attnb_ref, wh_ref, v_ref, dembT_ref, decW_ref,
                    bdec_ref, fcw_ref, fcb_ref, out_ref, es_ref, ps_ref,
                    ps2_ref):
    f32 = jnp.float32
    bf16 = jnp.bfloat16
    iota64 = jax.lax.broadcasted_iota(jnp.int32, (64, W_BLK), 0)
    h0 = jnp.zeros((HID, W_BLK), f32)

    def lstm_gates(g, c):
        i_ = jax.nn.sigmoid(g[0:64])
        f_ = jax.nn.sigmoid(g[64:128])
        g_ = jnp.tanh(g[128:192])
        o_ = jax.nn.sigmoid(g[192:256])
        c = f_ * c + i_ * g_
        h = o_ * jnp.tanh(c)
        return h, c

    def enc_body(t, carry):
        h, c = carry
        oh = (iota64 == src_ref[t]).astype(f32)
        x = jnp.dot(embT_ref[...], oh, preferred_element_type=f32)
        xh = jnp.concatenate([x, h], axis=0)
        g = jnp.dot(encW_ref[...], xh, preferred_element_type=f32) + benc_ref[...]
        h, c = lstm_gates(g, c)
        es_ref[t] = h.astype(bf16)
        ps_ref[t] = (jnp.dot(we_ref[...], h, preferred_element_type=f32)
                     + attnb_ref[...]).astype(bf16)
        return h, c

    h, c = jax.lax.fori_loop(0, S_SRC, enc_body, (h0, h0))

    for k0 in range(0, HID, 8):
        ps2_ref[k0:k0 + 8] = jnp.transpose(ps_ref[:, k0:k0 + 8, :], (1, 0, 2))

    def dec_body(t, carry):
        h, c, tok = carry
        oh = (iota64 == tok).astype(f32)
        e = jnp.dot(dembT_ref[...], oh, preferred_element_type=f32)
        q = jnp.dot(wh_ref[...], h, preferred_element_type=f32)
        v3 = v_ref[...]
        ctxs = []
        for lo in (0, HB):
            q3 = q[:, lo:lo + HB].reshape(HID, 1, HB).astype(bf16)
            sc = jnp.zeros((S_SRC, HB), bf16)
            for g8 in range(0, HID, 8):
                tm = [v3[g8 + j] * jnp.tanh(q3[g8 + j]
                                            + ps2_ref[g8 + j, :, lo:lo + HB])
                      for j in range(8)]
                sc = sc + (((tm[0] + tm[1]) + (tm[2] + tm[3]))
                           + ((tm[4] + tm[5]) + (tm[6] + tm[7])))
            scf = sc.astype(f32)
            m = jnp.max(scf, axis=0, keepdims=True)
            ex = jnp.exp(scf - m)
            l = jnp.sum(ex, axis=0, keepdims=True)
            a3 = (ex / l).reshape(S_SRC, 1, HB).astype(bf16)
            parts = []
            for j in range(4):
                acc = jnp.zeros((HID, HB), bf16)
                for s in range(j * 64, (j + 1) * 64):
                    acc = acc + a3[s] * es_ref[s, :, lo:lo + HB]
                parts.append(acc.astype(f32))
            ctxs.append((parts[0] + parts[1]) + (parts[2] + parts[3]))
        ctx = jnp.concatenate(ctxs, axis=1)
        x = jnp.concatenate([e, ctx, h], axis=0)
        g = jnp.dot(decW_ref[...], x, preferred_element_type=f32) + bdec_ref[...]
        h, c = lstm_gates(g, c)
        pred = jnp.dot(fcw_ref[...], h, preferred_element_type=f32) + fcb_ref[...]
        out_ref[t] = pred[0:OUT_DIM]
        mx = jnp.max(pred, axis=0, keepdims=True)
        tok = jnp.min(jnp.where(pred == mx, iota64, jnp.int32(63)),
                      axis=0, keepdims=True)
        return h, c, tok

    jax.lax.fori_loop(0, S_TRG - 1, dec_body, (h, c, tok0_ref[...]))


def kernel(src, trg, enc_emb, enc_Wih, enc_Whh, enc_bih, enc_bhh, attn_W,
           attn_b, v_w, dec_emb, dec_Wih, dec_Whh, dec_bih, dec_bhh,
           fc_W, fc_b):
    f32 = jnp.float32
    b = src.shape[0]
    srcT = src.T.reshape(S_SRC, 1, b)
    tok0 = trg[:, 0].reshape(1, b)
    embT = jnp.zeros((EMB, 64), f32).at[:, :IN_DIM].set(enc_emb.T)
    dembT = jnp.zeros((EMB, 64), f32).at[:, :OUT_DIM].set(dec_emb.T)
    encW = jnp.concatenate([enc_Wih, enc_Whh], axis=1)
    benc = jnp.broadcast_to((enc_bih + enc_bhh)[:, None], (G4, W_BLK))
    W_h, W_e = attn_W[:, :HID], attn_W[:, HID:]
    attnbB = jnp.broadcast_to(attn_b[:, None], (HID, W_BLK))
    vB = jnp.broadcast_to(
        v_w.astype(jnp.bfloat16)[:, None, None], (HID, 1, HB))
    decW = jnp.concatenate([dec_Wih, dec_Whh], axis=1)
    bdec = jnp.broadcast_to((dec_bih + dec_bhh)[:, None], (G4, W_BLK))
    fcW64 = jnp.zeros((64, HID), f32).at[:OUT_DIM].set(fc_W)
    fcb64 = jnp.full((64,), -1e30, f32).at[:OUT_DIM].set(fc_b)
    fcbB = jnp.broadcast_to(fcb64[:, None], (64, W_BLK))

    def full(shape):
        return pl.BlockSpec(shape, lambda i: tuple(0 for _ in shape))

    def run_blocks(srcT_l, tok0_l, *weights):
        b_l = srcT_l.shape[-1]
        grid = (b_l // W_BLK,)
        in_specs = [
            pl.BlockSpec((S_SRC, 1, W_BLK), lambda i: (0, 0, i)),
            pl.BlockSpec((1, W_BLK), lambda i: (0, i)),
            full((EMB, 64)), full((G4, 96)), full((G4, W_BLK)),
            full((HID, HID)), full((HID, W_BLK)), full((HID, HID)),
            full((HID, 1, HB)), full((EMB, 64)), full((G4, 160)),
            full((G4, W_BLK)), full((64, HID)), full((64, W_BLK)),
        ]
        out_specs = pl.BlockSpec((S_TRG - 1, OUT_DIM, W_BLK),
                                 lambda i: (0, 0, i))
        return pl.pallas_call(
            _seq2seq_kernel,
            out_shape=jax.ShapeDtypeStruct((S_TRG - 1, OUT_DIM, b_l), f32),
            grid=grid,
            in_specs=in_specs,
            out_specs=out_specs,
            scratch_shapes=[
                pltpu.VMEM((S_SRC, HID, W_BLK), jnp.bfloat16),
                pltpu.VMEM((S_SRC, HID, W_BLK), jnp.bfloat16),
                pltpu.VMEM((HID, S_SRC, W_BLK), jnp.bfloat16),
            ],
            compiler_params=pltpu.CompilerParams(
                dimension_semantics=("parallel",),
                vmem_limit_bytes=55 * 1024 * 1024,
            ),
            name="seq2seq_fused",
            interpret=_INTERPRET,
        )(srcT_l, tok0_l, *weights)

    args = (srcT, tok0, embT, encW, benc, W_e, attnbB, W_h, vB, dembT,
            decW, bdec, fcW64, fcbB)
    devs = jax.devices()
    n_dev = 2 if len(devs) >= 2 and b % (2 * W_BLK) == 0 else 1
    if n_dev == 2:
        mesh = Mesh(np.asarray(devs[:2]), ("d",))
        w_specs = tuple(P() for _ in range(len(args) - 2))
        predsT = jax.shard_map(
            run_blocks, mesh=mesh,
            in_specs=(P(None, None, "d"), P(None, "d")) + w_specs,
            out_specs=P(None, None, "d"),
            check_vma=False,
        )(*args)
    else:
        predsT = run_blocks(*args)
    preds = predsT.transpose(2, 0, 1)
    return jnp.concatenate([jnp.zeros((b, 1, OUT_DIM), f32), preds], axis=1)
